# baseline (speedup 1.0000x reference)
import jax
import jax.numpy as jnp
from jax.experimental import pallas as pl
from jax.experimental.pallas import tpu as pltpu


def _ngcn_kernel(adja_ref, adjb_ref, x_ref, wcat_ref, w3_ref, bcat_ref,
                 wfc_ref, bfc_ref, o_ref, y1_scr, y2_scr):
    p = pl.program_id(0)
    i = pl.program_id(1)
    bm = adja_ref.shape[0]

    @pl.when(p == 0)
    def _hop1():
        y1_scr[pl.ds(2 * i * bm, bm), :] = jnp.dot(
            adja_ref[...], x_ref[...], preferred_element_type=jnp.float32)
        y1_scr[pl.ds((2 * i + 1) * bm, bm), :] = jnp.dot(
            adjb_ref[...], x_ref[...], preferred_element_type=jnp.float32)

    @pl.when(p == 1)
    def _hop2():
        y2_scr[pl.ds(2 * i * bm, bm), :] = jnp.dot(
            adja_ref[...], y1_scr[...], preferred_element_type=jnp.float32)
        y2_scr[pl.ds((2 * i + 1) * bm, bm), :] = jnp.dot(
            adjb_ref[...], y1_scr[...], preferred_element_type=jnp.float32)

    @pl.when(p == 2)
    def _hop3_epilogue():
        for half, adj_ref in enumerate((adja_ref, adjb_ref)):
            r = (2 * i + half) * bm
            y3 = jnp.dot(adj_ref[...], y2_scr[...],
                         preferred_element_type=jnp.float32)
            y12 = jnp.concatenate(
                [y1_scr[pl.ds(r, bm), :], y2_scr[pl.ds(r, bm), :]], axis=1)
            h12 = jnp.dot(y12, wcat_ref[...],
                          preferred_element_type=jnp.float32)
            h3 = jnp.dot(y3, w3_ref[...], preferred_element_type=jnp.float32)
            h = jax.nn.relu(jnp.concatenate([h12, h3], axis=1) +
                            bcat_ref[...])
            logits = jnp.dot(h, wfc_ref[...],
                             preferred_element_type=jnp.float32)
            o_ref[pl.ds(half * bm, bm), :] = jax.nn.sigmoid(
                logits + bfc_ref[...])


def _pick_bm(m):
    for bm in (400, 200, 80, 40, 16, 8):
        if m % bm == 0:
            return bm
    return m


def kernel(x, adj, W1, b1, W2, b2, W3, b3, Wfc, bfc):
    m, n = adj.shape
    nh = W1.shape[1]
    nl = Wfc.shape[1]
    kh = Wfc.shape[0]
    bm = _pick_bm(m) // 2

    zeros = jnp.zeros_like(W1)
    wcat = jnp.block([[W1, zeros], [zeros, W2]])
    bcat = jnp.concatenate([b1, b2, b3])[None, :]

    return pl.pallas_call(
        _ngcn_kernel,
        grid=(3, m // (2 * bm)),
        in_specs=[
            pl.BlockSpec((bm, n), lambda p, i: (2 * i, 0)),
            pl.BlockSpec((bm, n), lambda p, i: (2 * i + 1, 0)),
            pl.BlockSpec((n, nh), lambda p, i: (0, 0)),
            pl.BlockSpec((2 * nh, 2 * nh), lambda p, i: (0, 0)),
            pl.BlockSpec((nh, nh), lambda p, i: (0, 0)),
            pl.BlockSpec((1, kh), lambda p, i: (0, 0)),
            pl.BlockSpec((kh, nl), lambda p, i: (0, 0)),
            pl.BlockSpec((1, nl), lambda p, i: (0, 0)),
        ],
        out_specs=pl.BlockSpec((2 * bm, nl), lambda p, i: (i, 0)),
        out_shape=jax.ShapeDtypeStruct((m, nl), jnp.float32),
        scratch_shapes=[
            pltpu.VMEM((m, nh), jnp.float32),
            pltpu.VMEM((m, nh), jnp.float32),
        ],
    )(adj, adj, x, wcat, W3, bcat, Wfc, bfc[None, :])
